# baseline (speedup 1.0000x reference)
import jax
import jax.numpy as jnp
from jax import lax
from jax.experimental import pallas as pl
from jax.experimental.pallas import tpu as pltpu
from jax.experimental.pallas import tpu_sc as plsc

B = 4096
L = 50
EMB = 64
T = B * L

NC = 2
NS = 16
NW = NC * NS
TPW = T // NW
G = 128
NG = TPW // G
NBUF = 4
SEG_REP = 512


def _emb_kernel(seq_hbm, lab_hbm, tok_hbm, seg_hbm, out_hbm,
                idx_v, lab_v, rows, seg_rows, gsem, ssem):
    wid = lax.axis_index("s") * NC + lax.axis_index("c")
    base = wid * TPW

    pltpu.sync_copy(seq_hbm.at[wid], idx_v)
    pltpu.sync_copy(lab_hbm.at[wid], lab_v)

    def gathers(g, b):
        return (
            pltpu.make_async_copy(tok_hbm.at[idx_v.at[g]], rows.at[b], gsem[b]),
            pltpu.make_async_copy(seg_hbm.at[lab_v.at[g]], seg_rows.at[b], gsem[b]),
        )

    def store(g, b):
        return pltpu.make_async_copy(
            rows.at[b], out_hbm.at[pl.ds(base + g * G, G)], ssem[b])

    def start_gathers(g, b):
        for d in gathers(g, b):
            d.start()

    def consume(g, b):
        for d in gathers(g, b):
            d.wait()

        def add_body(t, c):
            for q in range(EMB // 16):
                sl = pl.ds(q * 16, 16)
                plsc.addupdate(rows.at[b].at[t, sl], seg_rows[b, t, sl])
            return c

        lax.fori_loop(0, G, add_body, 0, unroll=4)
        store(g, b).start()

    start_gathers(0, 0)
    start_gathers(1, 1)
    start_gathers(2, 2)
    consume(0, 0)
    start_gathers(3, 3)
    consume(1, 1)
    store(0, 0).wait()
    start_gathers(4, 0)
    consume(2, 2)
    store(1, 1).wait()
    start_gathers(5, 1)
    consume(3, 3)

    def outer(o, carry):
        for i in range(NBUF):
            g = 4 + o * NBUF + i
            bpf = (i + 2) % NBUF
            store(g - 2, bpf).wait()
            start_gathers(g + 2, bpf)
            consume(g, i)
        return carry

    lax.fori_loop(0, (NG - 6) // NBUF, outer, 0)

    consume(NG - 2, (NG - 2) % NBUF)
    consume(NG - 1, (NG - 1) % NBUF)
    for g in range(NG - 4, NG):
        store(g, g % NBUF).wait()


TBLK = 8192


def _transpose_kernel(x_ref, o_ref):
    eye = jnp.eye(EMB, dtype=jnp.float32)
    o_ref[...] = jax.lax.dot_general(
        x_ref[...], eye, (((0,), (0,)), ((), ())),
        precision=jax.lax.Precision.HIGHEST,
        preferred_element_type=jnp.float32)


def _to_row_major(token_table):
    tt = token_table.T
    vocab = tt.shape[1]
    grid = (vocab + TBLK - 1) // TBLK
    return pl.pallas_call(
        _transpose_kernel,
        grid=(grid,),
        in_specs=[pl.BlockSpec((EMB, TBLK), lambda i: (0, i))],
        out_specs=pl.BlockSpec((TBLK, EMB), lambda i: (i, 0)),
        out_shape=jax.ShapeDtypeStruct((vocab, EMB), jnp.float32),
    )(tt)


@jax.jit
def _emb(seq_w, segidx_w, token_table, seg_rep):
    mesh = plsc.VectorSubcoreMesh(core_axis_name="c", subcore_axis_name="s")
    run = pl.kernel(
        _emb_kernel,
        out_type=jax.ShapeDtypeStruct((T, EMB), jnp.float32),
        mesh=mesh,
        scratch_types=[
            pltpu.VMEM((NG, G), jnp.int32),
            pltpu.VMEM((NG, G), jnp.int32),
            pltpu.VMEM((NBUF, G, EMB), jnp.float32),
            pltpu.VMEM((NBUF, G, EMB), jnp.float32),
            [pltpu.SemaphoreType.DMA] * NBUF,
            [pltpu.SemaphoreType.DMA] * NBUF,
        ],
        compiler_params=pltpu.CompilerParams(use_tc_tiling_on_sc=False),
    )
    return run(seq_w, segidx_w, _to_row_major(token_table), seg_rep)


def kernel(seq, segment_label, token_table, segment_table):
    seq_w = seq.reshape(NW, NG, G).astype(jnp.int32)
    seg_rep = jnp.tile(segment_table, (SEG_REP, 1))
    spread = (jnp.arange(T, dtype=jnp.int32) % SEG_REP) * 3
    segidx = segment_label.reshape(T).astype(jnp.int32) + spread
    segidx_w = segidx.reshape(NW, NG, G)
    out = _emb(seq_w, segidx_w, token_table, seg_rep)
    return out.reshape(B, L, EMB)


# speedup vs baseline: 1.2970x; 1.2970x over previous
import jax
import jax.numpy as jnp
from jax import lax
from jax.experimental import pallas as pl
from jax.experimental.pallas import tpu as pltpu
from jax.experimental.pallas import tpu_sc as plsc

B = 4096
L = 50
EMB = 64
T = B * L

NC = 2
NS = 16
NW = NC * NS
TPW = T // NW
G = 128
NG = TPW // G
NBUF = 6
PFD = 4
SEG_REP = 512


def _emb_kernel(seq_hbm, lab_hbm, tok_hbm, seg_hbm, out_hbm,
                idx_v, lab_v, rows, seg_rows, gsem, ssem):
    wid = lax.axis_index("s") * NC + lax.axis_index("c")
    base = wid * TPW

    pltpu.sync_copy(seq_hbm.at[wid], idx_v)
    pltpu.sync_copy(lab_hbm.at[wid], lab_v)

    def gathers(g, b):
        return (
            pltpu.make_async_copy(tok_hbm.at[idx_v.at[g]], rows.at[b], gsem[b]),
            pltpu.make_async_copy(seg_hbm.at[lab_v.at[g]], seg_rows.at[b], gsem[b]),
        )

    def store(g, b):
        return pltpu.make_async_copy(
            rows.at[b], out_hbm.at[pl.ds(base + g * G, G)], ssem[b])

    def start_gathers(g, b):
        for d in gathers(g, b):
            d.start()

    def consume(g, b):
        for d in gathers(g, b):
            d.wait()

        def add_body(t, c):
            for q in range(EMB // 16):
                sl = pl.ds(q * 16, 16)
                plsc.addupdate(rows.at[b].at[t, sl], seg_rows[b, t, sl])
            return c

        lax.fori_loop(0, G, add_body, 0, unroll=4)
        store(g, b).start()

    for g in range(PFD):
        start_gathers(g, g % NBUF)
    start_gathers(PFD, PFD % NBUF)
    consume(0, 0)
    start_gathers(PFD + 1, (PFD + 1) % NBUF)
    consume(1, 1)

    def outer(o, carry):
        for i in range(NBUF):
            g = 2 + o * NBUF + i
            bpf = i % NBUF
            store(g - 2, bpf).wait()
            start_gathers(g + PFD, bpf)
            consume(g, (2 + i) % NBUF)
        return carry

    lax.fori_loop(0, 7, outer, 0)

    store(42, 0).wait()
    start_gathers(48, 0)
    consume(44, 2)
    store(43, 1).wait()
    start_gathers(49, 1)
    consume(45, 3)
    consume(46, 4)
    consume(47, 5)
    consume(48, 0)
    consume(49, 1)
    for g in range(NG - NBUF, NG):
        store(g, g % NBUF).wait()


@jax.jit
def _emb(seq_w, segidx_w, token_table, seg_rep):
    mesh = plsc.VectorSubcoreMesh(core_axis_name="c", subcore_axis_name="s")
    run = pl.kernel(
        _emb_kernel,
        out_type=jax.ShapeDtypeStruct((T, EMB), jnp.float32),
        mesh=mesh,
        scratch_types=[
            pltpu.VMEM((NG, G), jnp.int32),
            pltpu.VMEM((NG, G), jnp.int32),
            pltpu.VMEM((NBUF, G, EMB), jnp.float32),
            pltpu.VMEM((NBUF, G, EMB), jnp.float32),
            [pltpu.SemaphoreType.DMA] * NBUF,
            [pltpu.SemaphoreType.DMA] * NBUF,
        ],
        compiler_params=pltpu.CompilerParams(use_tc_tiling_on_sc=False),
    )
    return run(seq_w, segidx_w, token_table, seg_rep)


def kernel(seq, segment_label, token_table, segment_table):
    seq_w = seq.reshape(NW, NG, G).astype(jnp.int32)
    seg_rep = jnp.tile(segment_table, (SEG_REP, 1))
    spread = (jnp.arange(T, dtype=jnp.int32) % SEG_REP) * 3
    segidx = segment_label.reshape(T).astype(jnp.int32) + spread
    segidx_w = segidx.reshape(NW, NG, G)
    out = _emb(seq_w, segidx_w, token_table, seg_rep)
    return out.reshape(B, L, EMB)


# speedup vs baseline: 1.3088x; 1.0091x over previous
import jax
import jax.numpy as jnp
from jax import lax
from jax.experimental import pallas as pl
from jax.experimental.pallas import tpu as pltpu
from jax.experimental.pallas import tpu_sc as plsc

B = 4096
L = 50
EMB = 64
T = B * L

NC = 2
NS = 16
NW = NC * NS
TPW = T // NW
G = 128
NG = TPW // G
NBUF = 6
PFD = 4
SEG_REP = 2048


def _emb_kernel(seq_hbm, lab_hbm, tok_hbm, seg_hbm, out_hbm,
                idx_v, lab_v, rows, seg_rows, gsem, ssem):
    wid = lax.axis_index("s") * NC + lax.axis_index("c")
    base = wid * TPW

    pltpu.sync_copy(seq_hbm.at[wid], idx_v)
    pltpu.sync_copy(lab_hbm.at[wid], lab_v)

    def gathers(g, b):
        return (
            pltpu.make_async_copy(tok_hbm.at[idx_v.at[g]], rows.at[b], gsem[b]),
            pltpu.make_async_copy(seg_hbm.at[lab_v.at[g]], seg_rows.at[b], gsem[b]),
        )

    def store(g, b):
        return pltpu.make_async_copy(
            rows.at[b], out_hbm.at[pl.ds(base + g * G, G)], ssem[b])

    def start_gathers(g, b):
        for d in gathers(g, b):
            d.start()

    def consume(g, b):
        for d in gathers(g, b):
            d.wait()

        def add_body(t, c):
            for q in range(EMB // 16):
                sl = pl.ds(q * 16, 16)
                plsc.addupdate(rows.at[b].at[t, sl], seg_rows[b, t, sl])
            return c

        lax.fori_loop(0, G, add_body, 0, unroll=4)
        store(g, b).start()

    for g in range(PFD):
        start_gathers(g, g % NBUF)
    start_gathers(PFD, PFD % NBUF)
    consume(0, 0)
    start_gathers(PFD + 1, (PFD + 1) % NBUF)
    consume(1, 1)

    def outer(o, carry):
        for i in range(NBUF):
            g = 2 + o * NBUF + i
            bpf = i % NBUF
            store(g - 2, bpf).wait()
            start_gathers(g + PFD, bpf)
            consume(g, (2 + i) % NBUF)
        return carry

    lax.fori_loop(0, 7, outer, 0)

    store(42, 0).wait()
    start_gathers(48, 0)
    consume(44, 2)
    store(43, 1).wait()
    start_gathers(49, 1)
    consume(45, 3)
    consume(46, 4)
    consume(47, 5)
    consume(48, 0)
    consume(49, 1)
    for g in range(NG - NBUF, NG):
        store(g, g % NBUF).wait()


@jax.jit
def _emb(seq_w, segidx_w, token_table, seg_rep):
    mesh = plsc.VectorSubcoreMesh(core_axis_name="c", subcore_axis_name="s")
    run = pl.kernel(
        _emb_kernel,
        out_type=jax.ShapeDtypeStruct((T, EMB), jnp.float32),
        mesh=mesh,
        scratch_types=[
            pltpu.VMEM((NG, G), jnp.int32),
            pltpu.VMEM((NG, G), jnp.int32),
            pltpu.VMEM((NBUF, G, EMB), jnp.float32),
            pltpu.VMEM((NBUF, G, EMB), jnp.float32),
            [pltpu.SemaphoreType.DMA] * NBUF,
            [pltpu.SemaphoreType.DMA] * NBUF,
        ],
        compiler_params=pltpu.CompilerParams(use_tc_tiling_on_sc=False),
    )
    return run(seq_w, segidx_w, token_table, seg_rep)


def kernel(seq, segment_label, token_table, segment_table):
    seq_w = seq.reshape(NW, NG, G).astype(jnp.int32)
    seg_rep = jnp.tile(segment_table, (SEG_REP, 1))
    spread = (jnp.arange(T, dtype=jnp.int32) % SEG_REP) * 3
    segidx = segment_label.reshape(T).astype(jnp.int32) + spread
    segidx_w = segidx.reshape(NW, NG, G)
    out = _emb(seq_w, segidx_w, token_table, seg_rep)
    return out.reshape(B, L, EMB)
